# baseline (speedup 1.0000x reference)
import functools

import jax
import jax.numpy as jnp
from jax import lax
from jax.experimental import pallas as pl
from jax.experimental.pallas import tpu as pltpu
from jax.experimental.pallas import tpu_sc as plsc

N = 10000
D = 128
E = 320000
NC = 2
NS = 16
NW = NC * NS
CH = 64
NBUF = 4
NCH = 160
EPW = NCH * CH
E_PAD = NW * EPW
N_ACC = 10112
SLAB = N_ACC // NS

_mesh = plsc.VectorSubcoreMesh(core_axis_name="c", subcore_axis_name="s")


@functools.partial(
    pl.kernel,
    out_type=jax.ShapeDtypeStruct((NC, N_ACC, D), jnp.float32),
    mesh=_mesh,
    scratch_types=(
        [pltpu.VMEM((NBUF, CH), jnp.int32) for _ in range(2)]
        + [pltpu.VMEM((CH, D), jnp.float32) for _ in range(NBUF)]
        + [pltpu.VMEM_SHARED((N_ACC, D), jnp.float32)]
        + [pltpu.SemaphoreType.DMA for _ in range(2 * NBUF)]
    ),
)
def _sc_segment_sum(h_hbm, src_hbm, dst_hbm, zeros_hbm, out_hbm,
                    src_ring, dst_ring, *rest):
    bufs = rest[:NBUF]
    acc = rest[NBUF]
    isems = rest[NBUF + 1:NBUF + 1 + NBUF]
    rsems = rest[NBUF + 1 + NBUF:]
    cid = lax.axis_index("c")
    sid = lax.axis_index("s")
    wid = sid * NC + cid
    ebase = wid * EPW

    def fetch_idx(jj, p):
        off = ebase + jj * CH
        pltpu.async_copy(src_hbm.at[pl.ds(off, CH)], src_ring.at[p], isems[p])
        pltpu.async_copy(dst_hbm.at[pl.ds(off, CH)], dst_ring.at[p], isems[p])

    def wait_idx(jj, p):
        off = ebase + jj * CH
        pltpu.make_async_copy(src_hbm.at[pl.ds(off, CH)], src_ring.at[p],
                              isems[p]).wait()
        pltpu.make_async_copy(dst_hbm.at[pl.ds(off, CH)], dst_ring.at[p],
                              isems[p]).wait()

    def gather_rows(p):
        pltpu.async_copy(h_hbm.at[src_ring.at[p]], bufs[p], rsems[p])

    def wait_rows(p):
        pltpu.make_async_copy(h_hbm.at[src_ring.at[p]], bufs[p],
                              rsems[p]).wait()

    for p in range(NBUF):
        fetch_idx(p, p)
    for p in range(NBUF - 1):
        wait_idx(p, p)
        gather_rows(p)
    base = sid * SLAB
    pltpu.sync_copy(zeros_hbm, acc.at[pl.ds(base, SLAB)])
    plsc.subcore_barrier()

    def body(i, carry):
        for b in range(NBUF):
            j = i * NBUF + b
            pdeep = (b + NBUF - 1) % NBUF

            @pl.when(j + NBUF - 1 < NCH)
            def _start_deep():
                wait_idx(j + NBUF - 1, pdeep)
                gather_rows(pdeep)

            wait_rows(b)
            pltpu.sync_copy(bufs[b], acc.at[dst_ring.at[b]], add=True)

            @pl.when(j + NBUF < NCH)
            def _prefetch_idx():
                fetch_idx(j + NBUF, b)

        return carry

    lax.fori_loop(0, NCH // NBUF, body, 0)
    plsc.subcore_barrier()
    pltpu.sync_copy(acc.at[pl.ds(base, SLAB)],
                    out_hbm.at[cid, pl.ds(base, SLAB)])


BN = 1000
NB = N // BN


def _mlp_body(relu_out, h_ref, p_ref, w1_ref, b1_ref, w2_ref, b2_ref, o_ref):
    hin = h_ref[...] + p_ref[0] + p_ref[1]
    z = jnp.dot(hin, w1_ref[...], preferred_element_type=jnp.float32)
    z = jnp.maximum(z + b1_ref[...], 0.0)
    y = jnp.dot(z, w2_ref[...], preferred_element_type=jnp.float32)
    y = y + b2_ref[...]
    if relu_out:
        y = jnp.maximum(y, 0.0)
    o_ref[...] = y


def _mlp(h, parts, W1, b1, W2, b2, relu_out):
    return pl.pallas_call(
        functools.partial(_mlp_body, relu_out),
        grid=(NB,),
        in_specs=[
            pl.BlockSpec((BN, D), lambda i: (i, 0)),
            pl.BlockSpec((NC, BN, D), lambda i: (0, i, 0)),
            pl.BlockSpec((D, D), lambda i: (0, 0)),
            pl.BlockSpec((1, D), lambda i: (0, 0)),
            pl.BlockSpec((D, D), lambda i: (0, 0)),
            pl.BlockSpec((1, D), lambda i: (0, 0)),
        ],
        out_specs=pl.BlockSpec((BN, D), lambda i: (i, 0)),
        out_shape=jax.ShapeDtypeStruct((N, D), jnp.float32),
    )(h, parts, W1, b1.reshape(1, D), W2, b2.reshape(1, D))


def kernel(x, adj_t, W1a, b1a, W2a, b2a, W1b, b1b, W2b, b2b,
           W1c, b1c, W2c, b2c):
    src = adj_t[0].astype(jnp.int32)
    dst = adj_t[1].astype(jnp.int32)
    pad = E_PAD - E
    src = jnp.concatenate([src, jnp.zeros((pad,), jnp.int32)])
    dst = jnp.concatenate([dst, jnp.full((pad,), N, jnp.int32)])
    zeros = jnp.zeros((SLAB, D), jnp.float32)
    h = x
    for (W1, b1, W2, b2, relu_out) in ((W1a, b1a, W2a, b2a, True),
                                       (W1b, b1b, W2b, b2b, True),
                                       (W1c, b1c, W2c, b2c, False)):
        parts = _sc_segment_sum(h, src, dst, zeros)
        h = _mlp(h, parts, W1, b1, W2, b2, relu_out)
    return h


# speedup vs baseline: 1.0202x; 1.0202x over previous
import functools

import jax
import jax.numpy as jnp
from jax import lax
from jax.experimental import pallas as pl
from jax.experimental.pallas import tpu as pltpu
from jax.experimental.pallas import tpu_sc as plsc

N = 10000
D = 128
E = 320000
NC = 2
NS = 16
NW = NC * NS
CH = 64
NBUF = 4
NCH = 160
EPW = NCH * CH
E_PAD = NW * EPW
N_ACC = 10112
SLAB = N_ACC // NS

_mesh = plsc.VectorSubcoreMesh(core_axis_name="c", subcore_axis_name="s")


@functools.partial(
    pl.kernel,
    out_type=jax.ShapeDtypeStruct((NC, N_ACC, D), jnp.float32),
    mesh=_mesh,
    scratch_types=(
        [pltpu.VMEM((NBUF, CH), jnp.int32) for _ in range(2)]
        + [pltpu.VMEM((CH, D), jnp.float32) for _ in range(NBUF)]
        + [pltpu.VMEM_SHARED((N_ACC, D), jnp.float32)]
        + [pltpu.SemaphoreType.DMA for _ in range(2 * NBUF)]
    ),
)
def _sc_segment_sum(h_hbm, src_hbm, dst_hbm, zeros_hbm, out_hbm,
                    src_ring, dst_ring, *rest):
    bufs = rest[:NBUF]
    acc = rest[NBUF]
    isems = rest[NBUF + 1:NBUF + 1 + NBUF]
    rsems = rest[NBUF + 1 + NBUF:]
    cid = lax.axis_index("c")
    sid = lax.axis_index("s")
    wid = sid * NC + cid
    ebase = wid * EPW

    def fetch_idx(jj, p):
        off = ebase + jj * CH
        pltpu.async_copy(src_hbm.at[pl.ds(off, CH)], src_ring.at[p], isems[p])
        pltpu.async_copy(dst_hbm.at[pl.ds(off, CH)], dst_ring.at[p], isems[p])

    def wait_idx(jj, p):
        off = ebase + jj * CH
        pltpu.make_async_copy(src_hbm.at[pl.ds(off, CH)], src_ring.at[p],
                              isems[p]).wait()
        pltpu.make_async_copy(dst_hbm.at[pl.ds(off, CH)], dst_ring.at[p],
                              isems[p]).wait()

    def gather_rows(p):
        pltpu.async_copy(h_hbm.at[src_ring.at[p]], bufs[p], rsems[p])

    def wait_rows(p):
        pltpu.make_async_copy(h_hbm.at[src_ring.at[p]], bufs[p],
                              rsems[p]).wait()

    for p in range(NBUF):
        fetch_idx(p, p)
    base = sid * SLAB
    pltpu.sync_copy(zeros_hbm, acc.at[pl.ds(base, SLAB)])
    for p in range(NBUF - 1):
        wait_idx(p, p)
        gather_rows(p)
    plsc.subcore_barrier()

    def body(i, carry):
        for b in range(NBUF):
            j = i * NBUF + b
            pdeep = (b + NBUF - 1) % NBUF

            @pl.when(j + NBUF - 1 < NCH)
            def _start_deep():
                wait_idx(j + NBUF - 1, pdeep)
                gather_rows(pdeep)

            wait_rows(b)
            pltpu.sync_copy(bufs[b], acc.at[dst_ring.at[b]], add=True)

            @pl.when(j + NBUF < NCH)
            def _prefetch_idx():
                fetch_idx(j + NBUF, b)

        return carry

    lax.fori_loop(0, NCH // NBUF, body, 0)
    plsc.subcore_barrier()
    pltpu.sync_copy(acc.at[pl.ds(base, SLAB)],
                    out_hbm.at[cid, pl.ds(base, SLAB)])


BN = 1000
NB = N // BN


def _mlp_body(relu_out, h_ref, p_ref, w1_ref, b1_ref, w2_ref, b2_ref, o_ref):
    hin = h_ref[...] + p_ref[0] + p_ref[1]
    z = jnp.dot(hin, w1_ref[...], preferred_element_type=jnp.float32)
    z = jnp.maximum(z + b1_ref[...], 0.0)
    y = jnp.dot(z, w2_ref[...], preferred_element_type=jnp.float32)
    y = y + b2_ref[...]
    if relu_out:
        y = jnp.maximum(y, 0.0)
    o_ref[...] = y


def _mlp(h, parts, W1, b1, W2, b2, relu_out):
    return pl.pallas_call(
        functools.partial(_mlp_body, relu_out),
        grid=(NB,),
        in_specs=[
            pl.BlockSpec((BN, D), lambda i: (i, 0)),
            pl.BlockSpec((NC, BN, D), lambda i: (0, i, 0)),
            pl.BlockSpec((D, D), lambda i: (0, 0)),
            pl.BlockSpec((1, D), lambda i: (0, 0)),
            pl.BlockSpec((D, D), lambda i: (0, 0)),
            pl.BlockSpec((1, D), lambda i: (0, 0)),
        ],
        out_specs=pl.BlockSpec((BN, D), lambda i: (i, 0)),
        out_shape=jax.ShapeDtypeStruct((N, D), jnp.float32),
    )(h, parts, W1, b1.reshape(1, D), W2, b2.reshape(1, D))


def kernel(x, adj_t, W1a, b1a, W2a, b2a, W1b, b1b, W2b, b2b,
           W1c, b1c, W2c, b2c):
    src = adj_t[0].astype(jnp.int32)
    dst = adj_t[1].astype(jnp.int32)
    pad = E_PAD - E
    src = jnp.concatenate([src, jnp.zeros((pad,), jnp.int32)])
    dst = jnp.concatenate([dst, jnp.full((pad,), N, jnp.int32)])
    zeros = jnp.zeros((SLAB, D), jnp.float32)
    h = x
    for (W1, b1, W2, b2, relu_out) in ((W1a, b1a, W2a, b2a, True),
                                       (W1b, b1b, W2b, b2b, True),
                                       (W1c, b1c, W2c, b2c, False)):
        parts = _sc_segment_sum(h, src, dst, zeros)
        h = _mlp(h, parts, W1, b1, W2, b2, relu_out)
    return h
